# baseline (speedup 1.0000x reference)
import jax
import jax.numpy as jnp
from jax import lax
from jax.experimental import pallas as pl
from jax.experimental.pallas import tpu as pltpu

B, N, D = 32, 2048, 768
DEPTH, HEADS, GRID, NC = 12, 12, 10, 40
T = GRID * GRID + 1
HD = D // HEADS
SCALE = HD ** -0.5
INTERVAL = 0.2
MLP_H = 4 * D
EPS = 1e-5
CELLS = GRID * GRID
CP = 128
TP = 128
CAT = 128 + D
NEG = -1e30


def _pillar_kernel(x_ref, w1a_ref, w1b_ref, w2_ref, w3_ref, wa1_ref, wa2_ref,
                   bninv_ref, shiftpos_ref, clspos_ref, tok_ref, seg_ref):
    xb = x_ref[0]
    c0 = xb[:, 0:1]
    c2 = xb[:, 2:3]
    iy = jnp.floor(jnp.clip(c0 + 1.0, 0.0, 1.99) / INTERVAL).astype(jnp.int32)
    ix = jnp.floor(jnp.clip(c2 + 1.0, 0.0, 1.99) / INTERVAL).astype(jnp.int32)
    cell = iy * GRID + ix

    lanes = lax.broadcasted_iota(jnp.int32, (N, CP), 1)
    onehot = (lanes == cell).astype(jnp.float32)
    ones = jnp.ones((N, 1), jnp.float32)
    cnt = lax.dot_general(onehot, ones, (((0,), (0,)), ((), ())),
                          preferred_element_type=jnp.float32)
    csum = lax.dot_general(onehot, xb, (((0,), (0,)), ((), ())),
                           precision=lax.Precision.HIGHEST,
                           preferred_element_type=jnp.float32)
    centroid = csum / jnp.maximum(cnt, 1.0)
    cg = jnp.dot(onehot, centroid, precision=lax.Precision.HIGHEST,
                 preferred_element_type=jnp.float32)
    diff = xb - cg

    h = jax.nn.relu(
        jnp.dot(xb, w1a_ref[...], preferred_element_type=jnp.float32)
        + jnp.dot(diff, w1b_ref[...], preferred_element_type=jnp.float32))
    h = jax.nn.relu(jnp.dot(h, w2_ref[...], preferred_element_type=jnp.float32))
    h = jax.nn.relu(jnp.dot(h, w3_ref[...], preferred_element_type=jnp.float32))
    u = jnp.dot(h, wa1_ref[...], preferred_element_type=jnp.float32)
    cat = jnp.concatenate([h, u], axis=1)

    seg_ref[...] = jnp.full((16, 8, CAT), NEG, jnp.float32)

    def loop_body(i, carry):
        vals = []
        for c8 in range(8):
            m = cell == (i * 8 + c8)
            vals.append(jnp.max(jnp.where(m, cat, NEG), axis=0, keepdims=True))
        seg_ref[pl.ds(i, 1)] = jnp.concatenate(vals, axis=0)[None]
        return carry

    lax.fori_loop(0, 13, loop_body, 0)

    seg = seg_ref[...].reshape(CP, CAT)
    pooled = jnp.maximum(seg[:, :128], 0.0)
    segu = seg[:, 128:]
    pillar = jax.nn.relu(
        segu + jnp.dot(pooled, wa2_ref[...], preferred_element_type=jnp.float32))
    out = pillar * bninv_ref[...] + shiftpos_ref[...]
    tok_ref[0] = jnp.concatenate([clspos_ref[...], out[:TP - 1]], axis=0)


def _ln(x, g, b):
    m = jnp.mean(x, axis=-1, keepdims=True)
    xc = x - m
    v = jnp.mean(xc * xc, axis=-1, keepdims=True)
    return xc * lax.rsqrt(v + EPS) * g + b


def _vit_kernel(tok_ref, g1_ref, b1_ref, qkv_ref, pw_ref, pb_ref,
                g2_ref, b2_ref, f1w_ref, f1b_ref, f2w_ref, f2b_ref, out_ref,
                tscr_ref):
    l = pl.program_id(1)
    b = pl.program_id(2)

    @pl.when(l == 0)
    def _():
        tscr_ref[b] = tok_ref[0]

    t = tscr_ref[b]
    y = _ln(t, g1_ref[0], b1_ref[0])
    qkv = jnp.dot(y.astype(jnp.bfloat16), qkv_ref[0],
                  preferred_element_type=jnp.float32)
    kmask = lax.broadcasted_iota(jnp.int32, (1, TP), 1) < T
    outs = []
    for hh in range(HEADS):
        q = (qkv[:, hh * HD:(hh + 1) * HD] * SCALE).astype(jnp.bfloat16)
        k = qkv[:, D + hh * HD:D + (hh + 1) * HD].astype(jnp.bfloat16)
        v = qkv[:, 2 * D + hh * HD:2 * D + (hh + 1) * HD].astype(jnp.bfloat16)
        s = lax.dot_general(q, k, (((1,), (1,)), ((), ())),
                            preferred_element_type=jnp.float32)
        s = jnp.where(kmask, s, NEG)
        e = jnp.exp(s - jnp.max(s, axis=1, keepdims=True))
        p = e / jnp.sum(e, axis=1, keepdims=True)
        outs.append(jnp.dot(p.astype(jnp.bfloat16), v,
                            preferred_element_type=jnp.float32))
    o = jnp.concatenate(outs, axis=1)
    t = t + jnp.dot(o.astype(jnp.bfloat16), pw_ref[0],
                    preferred_element_type=jnp.float32) + pb_ref[0]
    y = _ln(t, g2_ref[0], b2_ref[0])
    a = jnp.dot(y.astype(jnp.bfloat16), f1w_ref[0],
                preferred_element_type=jnp.float32) + f1b_ref[0]
    g = a * 0.5 * (1.0 + lax.erf(a * (2.0 ** -0.5)))
    t = t + jnp.dot(g.astype(jnp.bfloat16), f2w_ref[0],
                    preferred_element_type=jnp.float32) + f2b_ref[0]
    tscr_ref[b] = t
    out_ref[0] = t


def _head_kernel(cls_ref, g_ref, b_ref, hw_ref, hb_ref, out_ref):
    y = _ln(cls_ref[...], g_ref[...], b_ref[...])
    out_ref[...] = (jnp.dot(y, hw_ref[...], preferred_element_type=jnp.float32)
                    + hb_ref[...])


def kernel(x, W1, W2, W3, Wa, bn_g, bn_b, bn_mean, bn_var, cls_token, pos_embed,
           ln1_g, ln1_b, qkv_w, proj_w, proj_b, ln2_g, ln2_b,
           fc1_w, fc1_b, fc2_w, fc2_b, norm_g, norm_b, head_w, head_b):
    f32 = jnp.float32
    bf16 = jnp.bfloat16

    perm = jnp.array([1, 0, 2], dtype=jnp.int32)
    W1x = jnp.concatenate([W1[:3][perm], W1[3:][perm]], axis=0)
    w1a, w1b = W1x[:3], W1x[3:]
    wa1, wa2 = Wa[:128], Wa[128:]

    inv = bn_g * lax.rsqrt(bn_var + EPS)
    shift = bn_b - bn_mean * inv
    bninv = jnp.zeros((CP, D), f32).at[:CELLS].set(
        jnp.broadcast_to(inv[:, None], (CELLS, D)))
    shiftpos = jnp.zeros((CP, D), f32).at[:CELLS].set(
        shift[:, None] + pos_embed[0, 1:T])
    clspos = cls_token[0] + pos_embed[0, :1]

    tokens = pl.pallas_call(
        _pillar_kernel,
        grid=(B,),
        in_specs=[
            pl.BlockSpec((1, N, 3), lambda b: (b, 0, 0)),
            pl.BlockSpec((3, 32), lambda b: (0, 0)),
            pl.BlockSpec((3, 32), lambda b: (0, 0)),
            pl.BlockSpec((32, 64), lambda b: (0, 0)),
            pl.BlockSpec((64, 128), lambda b: (0, 0)),
            pl.BlockSpec((128, D), lambda b: (0, 0)),
            pl.BlockSpec((128, D), lambda b: (0, 0)),
            pl.BlockSpec((CP, D), lambda b: (0, 0)),
            pl.BlockSpec((CP, D), lambda b: (0, 0)),
            pl.BlockSpec((1, D), lambda b: (0, 0)),
        ],
        out_specs=pl.BlockSpec((1, TP, D), lambda b: (b, 0, 0)),
        out_shape=jax.ShapeDtypeStruct((B, TP, D), f32),
        scratch_shapes=[pltpu.VMEM((16, 8, CAT), f32)],
        compiler_params=pltpu.CompilerParams(
            dimension_semantics=("parallel",),
            vmem_limit_bytes=56 * 1024 * 1024,
        ),
        name="pillar_encode",
    )(x, w1a, w1b, W2, W3, wa1, wa2, bninv, shiftpos, clspos)

    bh = B // 2
    tokens = pl.pallas_call(
        _vit_kernel,
        grid=(2, DEPTH, bh),
        in_specs=[
            pl.BlockSpec((1, TP, D), lambda c, l, b: (c * bh + b, 0, 0)),
            pl.BlockSpec((1, 1, D), lambda c, l, b: (l, 0, 0)),
            pl.BlockSpec((1, 1, D), lambda c, l, b: (l, 0, 0)),
            pl.BlockSpec((1, D, 3 * D), lambda c, l, b: (l, 0, 0)),
            pl.BlockSpec((1, D, D), lambda c, l, b: (l, 0, 0)),
            pl.BlockSpec((1, 1, D), lambda c, l, b: (l, 0, 0)),
            pl.BlockSpec((1, 1, D), lambda c, l, b: (l, 0, 0)),
            pl.BlockSpec((1, 1, D), lambda c, l, b: (l, 0, 0)),
            pl.BlockSpec((1, D, MLP_H), lambda c, l, b: (l, 0, 0)),
            pl.BlockSpec((1, 1, MLP_H), lambda c, l, b: (l, 0, 0)),
            pl.BlockSpec((1, MLP_H, D), lambda c, l, b: (l, 0, 0)),
            pl.BlockSpec((1, 1, D), lambda c, l, b: (l, 0, 0)),
        ],
        out_specs=pl.BlockSpec((1, TP, D), lambda c, l, b: (c * bh + b, 0, 0)),
        out_shape=jax.ShapeDtypeStruct((B, TP, D), f32),
        input_output_aliases={0: 0},
        scratch_shapes=[pltpu.VMEM((bh, TP, D), f32)],
        compiler_params=pltpu.CompilerParams(
            dimension_semantics=("parallel", "arbitrary", "arbitrary"),
            vmem_limit_bytes=56 * 1024 * 1024,
        ),
        name="vit_stack",
    )(tokens, ln1_g.reshape(DEPTH, 1, D), ln1_b.reshape(DEPTH, 1, D),
      qkv_w.astype(bf16), proj_w.astype(bf16), proj_b.reshape(DEPTH, 1, D),
      ln2_g.reshape(DEPTH, 1, D), ln2_b.reshape(DEPTH, 1, D),
      fc1_w.astype(bf16), fc1_b.reshape(DEPTH, 1, MLP_H),
      fc2_w.astype(bf16), fc2_b.reshape(DEPTH, 1, D))

    cls = tokens[:, 0, :]
    out = pl.pallas_call(
        _head_kernel,
        grid=(2,),
        in_specs=[
            pl.BlockSpec((B // 2, D), lambda c: (c, 0)),
            pl.BlockSpec((1, D), lambda c: (0, 0)),
            pl.BlockSpec((1, D), lambda c: (0, 0)),
            pl.BlockSpec((D, NC), lambda c: (0, 0)),
            pl.BlockSpec((1, NC), lambda c: (0, 0)),
        ],
        out_specs=pl.BlockSpec((B // 2, NC), lambda c: (c, 0)),
        out_shape=jax.ShapeDtypeStruct((B, NC), f32),
        compiler_params=pltpu.CompilerParams(
            dimension_semantics=("parallel",),
        ),
        name="cls_head",
    )(cls, norm_g.reshape(1, D), norm_b.reshape(1, D), head_w,
      head_b.reshape(1, NC))
    return out


# speedup vs baseline: 1.4218x; 1.4218x over previous
import jax
import jax.numpy as jnp
from jax import lax
from jax.experimental import pallas as pl
from jax.experimental.pallas import tpu as pltpu

B, N, D = 32, 2048, 768
DEPTH, HEADS, GRID, NC = 12, 12, 10, 40
T = GRID * GRID + 1
HD = D // HEADS
SCALE = HD ** -0.5
INTERVAL = 0.2
MLP_H = 4 * D
EPS = 1e-5
CELLS = GRID * GRID
CP = 128
TP = 128
CAT = 128 + D
NEG = -1e30


def _pillar_kernel(x_ref, w1a_ref, w1b_ref, w2_ref, w3_ref, wa1_ref, wa2_ref,
                   bninv_ref, shiftpos_ref, clspos_ref, tok_ref, seg_ref):
    xb = x_ref[0]
    c0 = xb[:, 0:1]
    c2 = xb[:, 2:3]
    iy = jnp.floor(jnp.clip(c0 + 1.0, 0.0, 1.99) / INTERVAL).astype(jnp.int32)
    ix = jnp.floor(jnp.clip(c2 + 1.0, 0.0, 1.99) / INTERVAL).astype(jnp.int32)
    cell = iy * GRID + ix

    lanes = lax.broadcasted_iota(jnp.int32, (N, CP), 1)
    onehot = (lanes == cell).astype(jnp.float32)
    ones = jnp.ones((N, 1), jnp.float32)
    cnt = lax.dot_general(onehot, ones, (((0,), (0,)), ((), ())),
                          preferred_element_type=jnp.float32)
    csum = lax.dot_general(onehot, xb, (((0,), (0,)), ((), ())),
                           precision=lax.Precision.HIGHEST,
                           preferred_element_type=jnp.float32)
    centroid = csum / jnp.maximum(cnt, 1.0)
    cg = jnp.dot(onehot, centroid, precision=lax.Precision.HIGHEST,
                 preferred_element_type=jnp.float32)
    diff = xb - cg

    h = jax.nn.relu(
        jnp.dot(xb, w1a_ref[...], preferred_element_type=jnp.float32)
        + jnp.dot(diff, w1b_ref[...], preferred_element_type=jnp.float32))
    h = jax.nn.relu(jnp.dot(h, w2_ref[...], preferred_element_type=jnp.float32))
    h = jax.nn.relu(jnp.dot(h, w3_ref[...], preferred_element_type=jnp.float32))
    u = jnp.dot(h, wa1_ref[...], preferred_element_type=jnp.float32)
    cat = jnp.concatenate([h, u], axis=1)

    seg_ref[...] = jnp.full((16, 8, CAT), NEG, jnp.float32)

    def loop_body(i, carry):
        vals = []
        for c8 in range(8):
            m = cell == (i * 8 + c8)
            vals.append(jnp.max(jnp.where(m, cat, NEG), axis=0, keepdims=True))
        seg_ref[pl.ds(i, 1)] = jnp.concatenate(vals, axis=0)[None]
        return carry

    lax.fori_loop(0, 13, loop_body, 0)

    seg = seg_ref[...].reshape(CP, CAT)
    pooled = jnp.maximum(seg[:, :128], 0.0)
    segu = seg[:, 128:]
    pillar = jax.nn.relu(
        segu + jnp.dot(pooled, wa2_ref[...], preferred_element_type=jnp.float32))
    out = pillar * bninv_ref[...] + shiftpos_ref[...]
    tok_ref[0] = jnp.concatenate([clspos_ref[...], out[:TP - 1]], axis=0)


def _ln(x, g, b):
    m = jnp.mean(x, axis=-1, keepdims=True)
    xc = x - m
    v = jnp.mean(xc * xc, axis=-1, keepdims=True)
    return xc * lax.rsqrt(v + EPS) * g + b


VG = 2


def _vit_kernel(tok_ref, g1_ref, b1_ref, qkv_ref, pw_ref, pb_ref,
                g2_ref, b2_ref, f1w_ref, f1b_ref, f2w_ref, f2b_ref, out_ref,
                tscr_ref):
    l = pl.program_id(1)
    b = pl.program_id(2)

    @pl.when(l == 0)
    def _():
        tscr_ref[pl.ds(b * VG, VG)] = tok_ref[...]

    bf = jnp.bfloat16
    f32 = jnp.float32
    kmask = lax.broadcasted_iota(jnp.int32, (1, TP), 1) < T
    ts = [tscr_ref[b * VG + g] for g in range(VG)]
    ys = [_ln(t, g1_ref[0], b1_ref[0]).astype(bf) for t in ts]
    qkvs = [jnp.dot(y, qkv_ref[0], preferred_element_type=f32) for y in ys]
    ss = []
    for g in range(VG):
        for hh in range(HEADS):
            q = (qkvs[g][:, hh * HD:(hh + 1) * HD] * SCALE).astype(bf)
            k = qkvs[g][:, D + hh * HD:D + (hh + 1) * HD].astype(bf)
            ss.append(lax.dot_general(q, k, (((1,), (1,)), ((), ())),
                                      preferred_element_type=f32))
    ps = []
    for s in ss:
        e = jnp.where(kmask, jnp.exp(s), 0.0)
        ps.append((e / jnp.sum(e, axis=1, keepdims=True)).astype(bf))
    os_ = []
    for g in range(VG):
        heads = []
        for hh in range(HEADS):
            v = qkvs[g][:, 2 * D + hh * HD:2 * D + (hh + 1) * HD].astype(bf)
            heads.append(jnp.dot(ps[g * HEADS + hh], v,
                                 preferred_element_type=f32))
        os_.append(jnp.concatenate(heads, axis=1).astype(bf))
    ts = [ts[g] + jnp.dot(os_[g], pw_ref[0], preferred_element_type=f32)
          + pb_ref[0] for g in range(VG)]
    ys = [_ln(t, g2_ref[0], b2_ref[0]).astype(bf) for t in ts]
    aa = [jnp.dot(y, f1w_ref[0], preferred_element_type=f32) + f1b_ref[0]
          for y in ys]
    gg = [(a * 0.5 * (1.0 + lax.erf(a * (2.0 ** -0.5)))).astype(bf) for a in aa]
    ts = [ts[g] + jnp.dot(gg[g], f2w_ref[0], preferred_element_type=f32)
          + f2b_ref[0] for g in range(VG)]
    t = jnp.concatenate([x[None] for x in ts], axis=0)
    tscr_ref[pl.ds(b * VG, VG)] = t
    out_ref[...] = t


def _head_kernel(cls_ref, g_ref, b_ref, hw_ref, hb_ref, out_ref):
    y = _ln(cls_ref[...], g_ref[...], b_ref[...])
    out_ref[...] = (jnp.dot(y, hw_ref[...], preferred_element_type=jnp.float32)
                    + hb_ref[...])


def kernel(x, W1, W2, W3, Wa, bn_g, bn_b, bn_mean, bn_var, cls_token, pos_embed,
           ln1_g, ln1_b, qkv_w, proj_w, proj_b, ln2_g, ln2_b,
           fc1_w, fc1_b, fc2_w, fc2_b, norm_g, norm_b, head_w, head_b):
    f32 = jnp.float32
    bf16 = jnp.bfloat16

    perm = jnp.array([1, 0, 2], dtype=jnp.int32)
    W1x = jnp.concatenate([W1[:3][perm], W1[3:][perm]], axis=0)
    w1a, w1b = W1x[:3], W1x[3:]
    wa1, wa2 = Wa[:128], Wa[128:]

    inv = bn_g * lax.rsqrt(bn_var + EPS)
    shift = bn_b - bn_mean * inv
    bninv = jnp.zeros((CP, D), f32).at[:CELLS].set(
        jnp.broadcast_to(inv[:, None], (CELLS, D)))
    shiftpos = jnp.zeros((CP, D), f32).at[:CELLS].set(
        shift[:, None] + pos_embed[0, 1:T])
    clspos = cls_token[0] + pos_embed[0, :1]

    tokens = pl.pallas_call(
        _pillar_kernel,
        grid=(B,),
        in_specs=[
            pl.BlockSpec((1, N, 3), lambda b: (b, 0, 0)),
            pl.BlockSpec((3, 32), lambda b: (0, 0)),
            pl.BlockSpec((3, 32), lambda b: (0, 0)),
            pl.BlockSpec((32, 64), lambda b: (0, 0)),
            pl.BlockSpec((64, 128), lambda b: (0, 0)),
            pl.BlockSpec((128, D), lambda b: (0, 0)),
            pl.BlockSpec((128, D), lambda b: (0, 0)),
            pl.BlockSpec((CP, D), lambda b: (0, 0)),
            pl.BlockSpec((CP, D), lambda b: (0, 0)),
            pl.BlockSpec((1, D), lambda b: (0, 0)),
        ],
        out_specs=pl.BlockSpec((1, TP, D), lambda b: (b, 0, 0)),
        out_shape=jax.ShapeDtypeStruct((B, TP, D), f32),
        scratch_shapes=[pltpu.VMEM((16, 8, CAT), f32)],
        compiler_params=pltpu.CompilerParams(
            dimension_semantics=("parallel",),
            vmem_limit_bytes=56 * 1024 * 1024,
        ),
        name="pillar_encode",
    )(x, w1a, w1b, W2, W3, wa1, wa2, bninv, shiftpos, clspos)

    bh = B // 2
    tokens = pl.pallas_call(
        _vit_kernel,
        grid=(2, DEPTH, bh // VG),
        in_specs=[
            pl.BlockSpec((VG, TP, D), lambda c, l, b: (c * (bh // VG) + b, 0, 0)),
            pl.BlockSpec((1, 1, D), lambda c, l, b: (l, 0, 0)),
            pl.BlockSpec((1, 1, D), lambda c, l, b: (l, 0, 0)),
            pl.BlockSpec((1, D, 3 * D), lambda c, l, b: (l, 0, 0)),
            pl.BlockSpec((1, D, D), lambda c, l, b: (l, 0, 0)),
            pl.BlockSpec((1, 1, D), lambda c, l, b: (l, 0, 0)),
            pl.BlockSpec((1, 1, D), lambda c, l, b: (l, 0, 0)),
            pl.BlockSpec((1, 1, D), lambda c, l, b: (l, 0, 0)),
            pl.BlockSpec((1, D, MLP_H), lambda c, l, b: (l, 0, 0)),
            pl.BlockSpec((1, 1, MLP_H), lambda c, l, b: (l, 0, 0)),
            pl.BlockSpec((1, MLP_H, D), lambda c, l, b: (l, 0, 0)),
            pl.BlockSpec((1, 1, D), lambda c, l, b: (l, 0, 0)),
        ],
        out_specs=pl.BlockSpec((VG, TP, D), lambda c, l, b: (c * (bh // VG) + b, 0, 0)),
        out_shape=jax.ShapeDtypeStruct((B, TP, D), f32),
        input_output_aliases={0: 0},
        scratch_shapes=[pltpu.VMEM((bh, TP, D), f32)],
        compiler_params=pltpu.CompilerParams(
            dimension_semantics=("parallel", "arbitrary", "arbitrary"),
            vmem_limit_bytes=56 * 1024 * 1024,
        ),
        name="vit_stack",
    )(tokens, ln1_g.reshape(DEPTH, 1, D), ln1_b.reshape(DEPTH, 1, D),
      qkv_w.astype(bf16), proj_w.astype(bf16), proj_b.reshape(DEPTH, 1, D),
      ln2_g.reshape(DEPTH, 1, D), ln2_b.reshape(DEPTH, 1, D),
      fc1_w.astype(bf16), fc1_b.reshape(DEPTH, 1, MLP_H),
      fc2_w.astype(bf16), fc2_b.reshape(DEPTH, 1, D))

    cls = tokens[:, 0, :]
    out = pl.pallas_call(
        _head_kernel,
        grid=(2,),
        in_specs=[
            pl.BlockSpec((B // 2, D), lambda c: (c, 0)),
            pl.BlockSpec((1, D), lambda c: (0, 0)),
            pl.BlockSpec((1, D), lambda c: (0, 0)),
            pl.BlockSpec((D, NC), lambda c: (0, 0)),
            pl.BlockSpec((1, NC), lambda c: (0, 0)),
        ],
        out_specs=pl.BlockSpec((B // 2, NC), lambda c: (c, 0)),
        out_shape=jax.ShapeDtypeStruct((B, NC), f32),
        compiler_params=pltpu.CompilerParams(
            dimension_semantics=("parallel",),
        ),
        name="cls_head",
    )(cls, norm_g.reshape(1, D), norm_b.reshape(1, D), head_w,
      head_b.reshape(1, NC))
    return out


# speedup vs baseline: 1.7046x; 1.1989x over previous
import jax
import jax.numpy as jnp
from jax import lax
from jax.experimental import pallas as pl
from jax.experimental.pallas import tpu as pltpu

B, N, D = 32, 2048, 768
DEPTH, HEADS, GRID, NC = 12, 12, 10, 40
T = GRID * GRID + 1
HD = D // HEADS
SCALE = HD ** -0.5
INTERVAL = 0.2
MLP_H = 4 * D
EPS = 1e-5
CELLS = GRID * GRID
CP = 128
TP = 128
CAT = 128 + D
NEG = -1e30


def _pillar_kernel(x_ref, w1a_ref, w1b_ref, w2_ref, w3_ref, wa1_ref, wa2_ref,
                   bninv_ref, shiftpos_ref, clspos_ref, tok_ref, seg_ref):
    xb = x_ref[0]
    c0 = xb[:, 0:1]
    c2 = xb[:, 2:3]
    iy = jnp.floor(jnp.clip(c0 + 1.0, 0.0, 1.99) / INTERVAL).astype(jnp.int32)
    ix = jnp.floor(jnp.clip(c2 + 1.0, 0.0, 1.99) / INTERVAL).astype(jnp.int32)
    cell = iy * GRID + ix

    lanes = lax.broadcasted_iota(jnp.int32, (N, CP), 1)
    onehot = (lanes == cell).astype(jnp.float32)
    ones = jnp.ones((N, 1), jnp.float32)
    cnt = lax.dot_general(onehot, ones, (((0,), (0,)), ((), ())),
                          preferred_element_type=jnp.float32)
    csum = lax.dot_general(onehot, xb, (((0,), (0,)), ((), ())),
                           precision=lax.Precision.HIGHEST,
                           preferred_element_type=jnp.float32)
    centroid = csum / jnp.maximum(cnt, 1.0)
    cg = jnp.dot(onehot, centroid, precision=lax.Precision.HIGHEST,
                 preferred_element_type=jnp.float32)
    diff = xb - cg

    h = jax.nn.relu(
        jnp.dot(xb, w1a_ref[...], preferred_element_type=jnp.float32)
        + jnp.dot(diff, w1b_ref[...], preferred_element_type=jnp.float32))
    h = jax.nn.relu(jnp.dot(h, w2_ref[...], preferred_element_type=jnp.float32))
    h = jax.nn.relu(jnp.dot(h, w3_ref[...], preferred_element_type=jnp.float32))
    u = jnp.dot(h, wa1_ref[...], preferred_element_type=jnp.float32)
    cat = jnp.concatenate([h, u], axis=1).astype(jnp.bfloat16)

    seg_ref[...] = jnp.full((16, 8, CAT), NEG, jnp.bfloat16)

    def loop_body(i, carry):
        vals = []
        for c8 in range(8):
            m = cell == (i * 8 + c8)
            vals.append(jnp.max(jnp.where(m, cat, NEG), axis=0, keepdims=True))
        seg_ref[pl.ds(i, 1)] = jnp.concatenate(vals, axis=0)[None]
        return carry

    lax.fori_loop(0, 13, loop_body, 0)

    seg = seg_ref[...].reshape(CP, CAT)
    pooled = jnp.maximum(seg[:, :128], jnp.bfloat16(0.0))
    segu = seg[:, 128:].astype(jnp.float32)
    pillar = jax.nn.relu(
        segu + jnp.dot(pooled, wa2_ref[...].astype(jnp.bfloat16),
                       preferred_element_type=jnp.float32))
    out = pillar * bninv_ref[...] + shiftpos_ref[...]
    tok_ref[0] = jnp.concatenate([clspos_ref[...], out[:TP - 1]], axis=0)


def _ln(x, g, b):
    m = jnp.mean(x, axis=-1, keepdims=True)
    xc = x - m
    v = jnp.mean(xc * xc, axis=-1, keepdims=True)
    return xc * lax.rsqrt(v + EPS) * g + b


VG = 2


def _vit_kernel(tok_ref, g1_ref, b1_ref, qkv_ref, pw_ref, pb_ref,
                g2_ref, b2_ref, f1w_ref, f1b_ref, f2w_ref, f2b_ref, out_ref,
                tscr_ref):
    l = pl.program_id(1)
    b = pl.program_id(2)

    @pl.when(l == 0)
    def _():
        tscr_ref[pl.ds(b * VG, VG)] = tok_ref[...]

    bf = jnp.bfloat16
    f32 = jnp.float32
    kmask = lax.broadcasted_iota(jnp.int32, (1, TP), 1) < T
    ts = [tscr_ref[b * VG + g] for g in range(VG)]
    ys = [_ln(t, g1_ref[0], b1_ref[0]).astype(bf) for t in ts]
    qkvs = [jnp.dot(y, qkv_ref[0], preferred_element_type=f32) for y in ys]
    ss = []
    for g in range(VG):
        for hh in range(HEADS):
            q = (qkvs[g][:, hh * HD:(hh + 1) * HD] * SCALE).astype(bf)
            k = qkvs[g][:, D + hh * HD:D + (hh + 1) * HD].astype(bf)
            ss.append(lax.dot_general(q, k, (((1,), (1,)), ((), ())),
                                      preferred_element_type=f32))
    ps = []
    for s in ss:
        e = jnp.where(kmask, jnp.exp(s), 0.0)
        ps.append((e / jnp.sum(e, axis=1, keepdims=True)).astype(bf))
    os_ = []
    for g in range(VG):
        heads = []
        for hh in range(HEADS):
            v = qkvs[g][:, 2 * D + hh * HD:2 * D + (hh + 1) * HD].astype(bf)
            heads.append(jnp.dot(ps[g * HEADS + hh], v,
                                 preferred_element_type=f32))
        os_.append(jnp.concatenate(heads, axis=1).astype(bf))
    ts = [ts[g] + jnp.dot(os_[g], pw_ref[0], preferred_element_type=f32)
          + pb_ref[0] for g in range(VG)]
    ys = [_ln(t, g2_ref[0], b2_ref[0]).astype(bf) for t in ts]
    aa = [jnp.dot(y, f1w_ref[0], preferred_element_type=f32) + f1b_ref[0]
          for y in ys]
    gg = [(a * 0.5 * (1.0 + lax.erf(a * (2.0 ** -0.5)))).astype(bf) for a in aa]
    ts = [ts[g] + jnp.dot(gg[g], f2w_ref[0], preferred_element_type=f32)
          + f2b_ref[0] for g in range(VG)]
    t = jnp.concatenate([x[None] for x in ts], axis=0)
    tscr_ref[pl.ds(b * VG, VG)] = t
    out_ref[...] = t


def _head_kernel(cls_ref, g_ref, b_ref, hw_ref, hb_ref, out_ref):
    y = _ln(cls_ref[...], g_ref[...], b_ref[...])
    out_ref[...] = (jnp.dot(y, hw_ref[...], preferred_element_type=jnp.float32)
                    + hb_ref[...])


def kernel(x, W1, W2, W3, Wa, bn_g, bn_b, bn_mean, bn_var, cls_token, pos_embed,
           ln1_g, ln1_b, qkv_w, proj_w, proj_b, ln2_g, ln2_b,
           fc1_w, fc1_b, fc2_w, fc2_b, norm_g, norm_b, head_w, head_b):
    f32 = jnp.float32
    bf16 = jnp.bfloat16

    perm = jnp.array([1, 0, 2], dtype=jnp.int32)
    W1x = jnp.concatenate([W1[:3][perm], W1[3:][perm]], axis=0)
    w1a, w1b = W1x[:3], W1x[3:]
    wa1, wa2 = Wa[:128], Wa[128:]

    inv = bn_g * lax.rsqrt(bn_var + EPS)
    shift = bn_b - bn_mean * inv
    bninv = jnp.zeros((CP, D), f32).at[:CELLS].set(
        jnp.broadcast_to(inv[:, None], (CELLS, D)))
    shiftpos = jnp.zeros((CP, D), f32).at[:CELLS].set(
        shift[:, None] + pos_embed[0, 1:T])
    clspos = cls_token[0] + pos_embed[0, :1]

    tokens = pl.pallas_call(
        _pillar_kernel,
        grid=(B,),
        in_specs=[
            pl.BlockSpec((1, N, 3), lambda b: (b, 0, 0)),
            pl.BlockSpec((3, 32), lambda b: (0, 0)),
            pl.BlockSpec((3, 32), lambda b: (0, 0)),
            pl.BlockSpec((32, 64), lambda b: (0, 0)),
            pl.BlockSpec((64, 128), lambda b: (0, 0)),
            pl.BlockSpec((128, D), lambda b: (0, 0)),
            pl.BlockSpec((128, D), lambda b: (0, 0)),
            pl.BlockSpec((CP, D), lambda b: (0, 0)),
            pl.BlockSpec((CP, D), lambda b: (0, 0)),
            pl.BlockSpec((1, D), lambda b: (0, 0)),
        ],
        out_specs=pl.BlockSpec((1, TP, D), lambda b: (b, 0, 0)),
        out_shape=jax.ShapeDtypeStruct((B, TP, D), f32),
        scratch_shapes=[pltpu.VMEM((16, 8, CAT), jnp.bfloat16)],
        compiler_params=pltpu.CompilerParams(
            dimension_semantics=("parallel",),
            vmem_limit_bytes=56 * 1024 * 1024,
        ),
        name="pillar_encode",
    )(x, w1a, w1b, W2, W3, wa1, wa2, bninv, shiftpos, clspos)

    bh = B // 2
    tokens = pl.pallas_call(
        _vit_kernel,
        grid=(2, DEPTH, bh // VG),
        in_specs=[
            pl.BlockSpec((VG, TP, D), lambda c, l, b: (c * (bh // VG) + b, 0, 0)),
            pl.BlockSpec((1, 1, D), lambda c, l, b: (l, 0, 0)),
            pl.BlockSpec((1, 1, D), lambda c, l, b: (l, 0, 0)),
            pl.BlockSpec((1, D, 3 * D), lambda c, l, b: (l, 0, 0)),
            pl.BlockSpec((1, D, D), lambda c, l, b: (l, 0, 0)),
            pl.BlockSpec((1, 1, D), lambda c, l, b: (l, 0, 0)),
            pl.BlockSpec((1, 1, D), lambda c, l, b: (l, 0, 0)),
            pl.BlockSpec((1, 1, D), lambda c, l, b: (l, 0, 0)),
            pl.BlockSpec((1, D, MLP_H), lambda c, l, b: (l, 0, 0)),
            pl.BlockSpec((1, 1, MLP_H), lambda c, l, b: (l, 0, 0)),
            pl.BlockSpec((1, MLP_H, D), lambda c, l, b: (l, 0, 0)),
            pl.BlockSpec((1, 1, D), lambda c, l, b: (l, 0, 0)),
        ],
        out_specs=pl.BlockSpec((VG, TP, D), lambda c, l, b: (c * (bh // VG) + b, 0, 0)),
        out_shape=jax.ShapeDtypeStruct((B, TP, D), f32),
        input_output_aliases={0: 0},
        scratch_shapes=[pltpu.VMEM((bh, TP, D), f32)],
        compiler_params=pltpu.CompilerParams(
            dimension_semantics=("parallel", "arbitrary", "arbitrary"),
            vmem_limit_bytes=56 * 1024 * 1024,
        ),
        name="vit_stack",
    )(tokens, ln1_g.reshape(DEPTH, 1, D), ln1_b.reshape(DEPTH, 1, D),
      qkv_w.astype(bf16), proj_w.astype(bf16), proj_b.reshape(DEPTH, 1, D),
      ln2_g.reshape(DEPTH, 1, D), ln2_b.reshape(DEPTH, 1, D),
      fc1_w.astype(bf16), fc1_b.reshape(DEPTH, 1, MLP_H),
      fc2_w.astype(bf16), fc2_b.reshape(DEPTH, 1, D))

    cls = tokens[:, 0, :]
    out = pl.pallas_call(
        _head_kernel,
        grid=(2,),
        in_specs=[
            pl.BlockSpec((B // 2, D), lambda c: (c, 0)),
            pl.BlockSpec((1, D), lambda c: (0, 0)),
            pl.BlockSpec((1, D), lambda c: (0, 0)),
            pl.BlockSpec((D, NC), lambda c: (0, 0)),
            pl.BlockSpec((1, NC), lambda c: (0, 0)),
        ],
        out_specs=pl.BlockSpec((B // 2, NC), lambda c: (c, 0)),
        out_shape=jax.ShapeDtypeStruct((B, NC), f32),
        compiler_params=pltpu.CompilerParams(
            dimension_semantics=("parallel",),
        ),
        name="cls_head",
    )(cls, norm_g.reshape(1, D), norm_b.reshape(1, D), head_w,
      head_b.reshape(1, NC))
    return out
